# Initial kernel scaffold; baseline (speedup 1.0000x reference)
import jax
import jax.numpy as jnp
from jax.experimental import pallas as pl


def kernel(x, edge_index, We, be, Wa, ba):
    raise NotImplementedError("write your pallas kernel here")


# speedup vs baseline: 4.8704x; 4.8704x over previous
import functools

import jax
import jax.numpy as jnp
from jax import lax
from jax.experimental import pallas as pl
from jax.experimental.pallas import tpu as pltpu
from jax.experimental.pallas import tpu_sc as plsc

C = 256
OUT = 256
K = 16
N = 10000
NPAD = 10240
LANES = 16

NC, NS = 2, 16
NW = NC * NS
NODES_PER_W = NPAD // NW
G = 4
ITERS = NODES_PER_W // G
ROWS = G * K


def _proj_body(x_ref, w_ref, b_ref, ti_ref, tj_ref):
    p = lax.dot_general(x_ref[...], w_ref[...], (((1,), (0,)), ((), ())),
                        preferred_element_type=jnp.float32)
    p = p + b_ref[...]
    ti_ref[...] = p[:, : 2 * OUT]
    tj_ref[...] = p[:, 2 * OUT:]


def _project(x2p, wc, bias2d):
    bn = 1024
    return pl.pallas_call(
        _proj_body,
        grid=(NPAD // bn,),
        in_specs=[
            pl.BlockSpec((bn, C), lambda i: (i, 0)),
            pl.BlockSpec((C, 4 * OUT), lambda i: (0, 0)),
            pl.BlockSpec((1, 4 * OUT), lambda i: (0, 0)),
        ],
        out_specs=[
            pl.BlockSpec((bn, 2 * OUT), lambda i: (i, 0)),
            pl.BlockSpec((bn, 2 * OUT), lambda i: (i, 0)),
        ],
        out_shape=[jax.ShapeDtypeStruct((NPAD, 2 * OUT), jnp.float32)] * 2,
    )(x2p, wc, bias2d)


def _sc_body(ti_hbm, tj_hbm, ii_hbm, jj_hbm, out_hbm,
             ii_v, jj_v, gi_v, gj_v, ob_v, sem_i, sem_j):
    wid = lax.axis_index("s") * NC + lax.axis_index("c")

    def iter_body(it, carry):
        base_node = wid * NODES_PER_W + it * G
        base_edge = base_node * K
        pltpu.sync_copy(ii_hbm.at[pl.ds(base_edge, ROWS)], ii_v)
        pltpu.sync_copy(jj_hbm.at[pl.ds(base_edge, ROWS)], jj_v)
        cp_i = pltpu.async_copy(ti_hbm.at[ii_v], gi_v, sem_i)
        cp_j = pltpu.async_copy(tj_hbm.at[jj_v], gj_v, sem_j)
        cp_i.wait()
        cp_j.wait()

        def compute(t, c2):
            g = t // LANES
            ob = t % LANES
            col_e = ob * LANES
            col_a = OUT + col_e
            r0 = g * K
            a = [gi_v[r0 + k, pl.ds(col_a, LANES)]
                 + gj_v[r0 + k, pl.ds(col_a, LANES)] for k in range(K)]
            m = a[0]
            for k in range(1, K):
                m = jnp.maximum(m, a[k])
            s = jnp.zeros((LANES,), jnp.float32)
            acc = jnp.zeros((LANES,), jnp.float32)
            for k in range(K):
                p = jnp.exp(a[k] - m)
                e = (gi_v[r0 + k, pl.ds(col_e, LANES)]
                     + gj_v[r0 + k, pl.ds(col_e, LANES)])
                s = s + p
                acc = acc + p * e
            ob_v[g, pl.ds(col_e, LANES)] = acc / s
            return c2

        lax.fori_loop(0, G * LANES, compute, 0)
        pltpu.sync_copy(ob_v, out_hbm.at[pl.ds(base_node, G)])
        return carry

    lax.fori_loop(0, ITERS, iter_body, 0)


def _edge_attn_sc(ti, tj, ii, jj):
    mesh = plsc.VectorSubcoreMesh(core_axis_name="c", subcore_axis_name="s")
    kfn = pl.kernel(
        _sc_body,
        out_type=jax.ShapeDtypeStruct((NPAD, OUT), jnp.float32),
        mesh=mesh,
        scratch_types=[
            pltpu.VMEM((ROWS,), jnp.int32),
            pltpu.VMEM((ROWS,), jnp.int32),
            pltpu.VMEM((ROWS, 2 * OUT), jnp.float32),
            pltpu.VMEM((ROWS, 2 * OUT), jnp.float32),
            pltpu.VMEM((G, OUT), jnp.float32),
            pltpu.SemaphoreType.DMA,
            pltpu.SemaphoreType.DMA,
        ],
    )
    return kfn(ti, tj, ii, jj)


def kernel(x, edge_index, We, be, Wa, ba):
    x2 = x[0, :, :, 0].T
    x2p = jnp.pad(x2, ((0, NPAD - N), (0, 0)))
    WeA, WeB = We[:, :C], We[:, C:]
    WaA, WaB = Wa[:, :C], Wa[:, C:]
    wc = jnp.concatenate(
        [(WeA - WeB).T, (WaA - WaB).T, WeB.T, WaB.T], axis=1)
    bias2d = jnp.concatenate(
        [be, jnp.zeros((3 * OUT,), jnp.float32)])[None, :]
    ii = jnp.pad(edge_index[1, 0], ((0, NPAD - N), (0, 0))).reshape(-1)
    jj = jnp.pad(edge_index[0, 0], ((0, NPAD - N), (0, 0))).reshape(-1)

    ti, tj = _project(x2p, wc, bias2d)

    out = _edge_attn_sc(ti, tj, ii, jj)

    return out[:N].T[None, :, :, None]


# speedup vs baseline: 9.4913x; 1.9488x over previous
import functools

import jax
import jax.numpy as jnp
from jax import lax
from jax.experimental import pallas as pl
from jax.experimental.pallas import tpu as pltpu
from jax.experimental.pallas import tpu_sc as plsc

C = 256
OUT = 256
K = 16
N = 10000
NPAD = 10240
LANES = 16

NC, NS = 2, 16
NW = NC * NS
NODES_PER_W = NPAD // NW
G = 4
ITERS = NODES_PER_W // G
ROWS = G * K


def _tree_reduce(op, xs):
    while len(xs) > 1:
        xs = [op(xs[i], xs[i + 1]) if i + 1 < len(xs) else xs[i]
              for i in range(0, len(xs), 2)]
    return xs[0]


def _proj_body(x_ref, w_ref, b_ref, ti_ref, tj_ref):
    p = lax.dot_general(x_ref[...], w_ref[...], (((1,), (0,)), ((), ())),
                        preferred_element_type=jnp.float32)
    p = p + b_ref[...]

    def pack(e, a):
        e16 = lax.bitcast_convert_type(e.astype(jnp.bfloat16), jnp.uint16)
        a16 = lax.bitcast_convert_type(a.astype(jnp.bfloat16), jnp.uint16)
        return e16.astype(jnp.uint32) | (a16.astype(jnp.uint32) << 16)

    ti_ref[...] = pack(p[:, :OUT], p[:, OUT:2 * OUT])
    tj_ref[...] = pack(p[:, 2 * OUT:3 * OUT], p[:, 3 * OUT:])


def _project(x2p, wc, bias2d):
    bn = 1024
    return pl.pallas_call(
        _proj_body,
        grid=(NPAD // bn,),
        in_specs=[
            pl.BlockSpec((bn, C), lambda i: (i, 0)),
            pl.BlockSpec((C, 4 * OUT), lambda i: (0, 0)),
            pl.BlockSpec((1, 4 * OUT), lambda i: (0, 0)),
        ],
        out_specs=[
            pl.BlockSpec((bn, OUT), lambda i: (i, 0)),
            pl.BlockSpec((bn, OUT), lambda i: (i, 0)),
        ],
        out_shape=[jax.ShapeDtypeStruct((NPAD, OUT), jnp.uint32)] * 2,
    )(x2p, wc, bias2d)


def _sc_body(ti_hbm, tj_hbm, ii_hbm, jj_hbm, out_hbm,
             ii_all, jj_all, gi0, gj0, gi1, gj1, ob_v,
             s_i0, s_j0, s_i1, s_j1):
    wid = lax.axis_index("s") * NC + lax.axis_index("c")
    base_w_edge = wid * NODES_PER_W * K
    pltpu.sync_copy(ii_hbm.at[pl.ds(base_w_edge, NODES_PER_W * K)], ii_all)
    pltpu.sync_copy(jj_hbm.at[pl.ds(base_w_edge, NODES_PER_W * K)], jj_all)

    def fire(it, gi_v, gj_v, sem_i, sem_j):
        off = it * ROWS
        pltpu.async_copy(ti_hbm.at[ii_all.at[pl.ds(off, ROWS)]], gi_v, sem_i)
        pltpu.async_copy(tj_hbm.at[jj_all.at[pl.ds(off, ROWS)]], gj_v, sem_j)

    def wait(gi_v, gj_v, sem_i, sem_j):
        pltpu.make_async_copy(ti_hbm.at[pl.ds(0, ROWS)], gi_v, sem_i).wait()
        pltpu.make_async_copy(tj_hbm.at[pl.ds(0, ROWS)], gj_v, sem_j).wait()

    hi_mask = jnp.full((LANES,), 0xFFFF0000, jnp.uint32)

    def compute(it, gi_v, gj_v):
        base_node = wid * NODES_PER_W + it * G

        def node_body(g, c0):
            r0 = g * K

            def ob_body(ob, c1):
                col = ob * LANES
                es = []
                ats = []
                for k in range(K):
                    wi = gi_v[r0 + k, pl.ds(col, LANES)]
                    wj = gj_v[r0 + k, pl.ds(col, LANES)]
                    ats.append(plsc.bitcast(wi & hi_mask, jnp.float32)
                               + plsc.bitcast(wj & hi_mask, jnp.float32))
                    es.append(plsc.bitcast(wi << 16, jnp.float32)
                              + plsc.bitcast(wj << 16, jnp.float32))
                m = _tree_reduce(jnp.maximum, ats)
                ps = [jnp.exp(a - m) for a in ats]
                s = _tree_reduce(lax.add, ps)
                acc = _tree_reduce(lax.add,
                                   [p * e for p, e in zip(ps, es)])
                ob_v[g, pl.ds(col, LANES)] = acc / s
                return c1

            lax.fori_loop(0, OUT // LANES, ob_body, 0)
            return c0

        lax.fori_loop(0, G, node_body, 0)
        pltpu.sync_copy(ob_v, out_hbm.at[pl.ds(base_node, G)])

    fire(0, gi0, gj0, s_i0, s_j0)

    def pair_body(ih, carry):
        it0 = ih * 2
        fire(it0 + 1, gi1, gj1, s_i1, s_j1)
        wait(gi0, gj0, s_i0, s_j0)
        compute(it0, gi0, gj0)

        @pl.when(it0 + 2 < ITERS)
        def _():
            fire(it0 + 2, gi0, gj0, s_i0, s_j0)

        wait(gi1, gj1, s_i1, s_j1)
        compute(it0 + 1, gi1, gj1)
        return carry

    lax.fori_loop(0, ITERS // 2, pair_body, 0)


def _edge_attn_sc(ti, tj, ii, jj):
    mesh = plsc.VectorSubcoreMesh(core_axis_name="c", subcore_axis_name="s")
    kfn = pl.kernel(
        _sc_body,
        out_type=jax.ShapeDtypeStruct((NPAD, OUT), jnp.float32),
        mesh=mesh,
        scratch_types=[
            pltpu.VMEM((NODES_PER_W * K,), jnp.int32),
            pltpu.VMEM((NODES_PER_W * K,), jnp.int32),
            pltpu.VMEM((ROWS, OUT), jnp.uint32),
            pltpu.VMEM((ROWS, OUT), jnp.uint32),
            pltpu.VMEM((ROWS, OUT), jnp.uint32),
            pltpu.VMEM((ROWS, OUT), jnp.uint32),
            pltpu.VMEM((G, OUT), jnp.float32),
            pltpu.SemaphoreType.DMA,
            pltpu.SemaphoreType.DMA,
            pltpu.SemaphoreType.DMA,
            pltpu.SemaphoreType.DMA,
        ],
        compiler_params=pltpu.CompilerParams(needs_layout_passes=False),
    )
    return kfn(ti, tj, ii, jj)


def kernel(x, edge_index, We, be, Wa, ba):
    x2 = x[0, :, :, 0].T
    x2p = jnp.pad(x2, ((0, NPAD - N), (0, 0)))
    WeA, WeB = We[:, :C], We[:, C:]
    WaA, WaB = Wa[:, :C], Wa[:, C:]
    wc = jnp.concatenate(
        [(WeA - WeB).T, (WaA - WaB).T, WeB.T, WaB.T], axis=1)
    bias2d = jnp.concatenate(
        [be, jnp.zeros((3 * OUT,), jnp.float32)])[None, :]
    ii = jnp.pad(edge_index[1, 0], ((0, NPAD - N), (0, 0))).reshape(-1)
    jj = jnp.pad(edge_index[0, 0], ((0, NPAD - N), (0, 0))).reshape(-1)

    ti, tj = _project(x2p, wc, bias2d)

    out = _edge_attn_sc(ti, tj, ii, jj)

    return out[:N].T[None, :, :, None]
